# baseline (speedup 1.0000x reference)
import jax
import jax.numpy as jnp
from jax import lax
from jax.experimental import pallas as pl
from jax.experimental.pallas import tpu as pltpu

B = 4
SEQ = 8192
D_MODEL = 4096
HIDDEN = 4096
NUM_LAYERS = 16

S_BLK = 256
H_BLK = 256
NS = SEQ // S_BLK
NH = HIDDEN // H_BLK
GRID = NS + NH + 1


def _router_kernel(x_ref, w1_ref, b1_ref, w2_ref, b2_ref, w3_ref, b3_ref,
                   logits_ref, idx_ref, acc_ref, h2_ref):
    i = pl.program_id(0)

    @pl.when(i == 0)
    def _init():
        acc_ref[...] = jnp.sum(x_ref[...], axis=1)

    @pl.when((i > 0) & (i < NS))
    def _pool():
        acc_ref[...] += jnp.sum(x_ref[...], axis=1)

    @pl.when((i >= NS) & (i < NS + NH))
    def _mlp():
        j = i - NS
        xp = acc_ref[...] * (1.0 / SEQ)
        pre1 = lax.dot_general(xp, w1_ref[...], (((1,), (1,)), ((), ())),
                               preferred_element_type=jnp.float32)
        h1 = jax.nn.gelu(pre1 + b1_ref[0])
        part = lax.dot_general(h1, w2_ref[...], (((1,), (1,)), ((), ())),
                               preferred_element_type=jnp.float32)

        @pl.when(j == 0)
        def _set():
            h2_ref[...] = part

        @pl.when(j > 0)
        def _add():
            h2_ref[...] += part

    @pl.when(i == NS + NH)
    def _final():
        h2 = jax.nn.gelu(h2_ref[...] + b2_ref[...])
        logits = lax.dot_general(h2, w3_ref[...], (((1,), (1,)), ((), ())),
                                 preferred_element_type=jnp.float32)
        logits = logits + b3_ref[...]
        logits_ref[...] = logits
        col = lax.broadcasted_iota(jnp.int32, (B, NUM_LAYERS), 1)
        maxv = jnp.max(logits, axis=1, keepdims=True)
        idx_ref[...] = jnp.min(
            jnp.where(logits == maxv, col, NUM_LAYERS), axis=1, keepdims=True)


def kernel(x, W1, b1, W2, b2, W3, b3):
    b1r = b1.reshape(NH, 1, H_BLK)
    b2r = b2.reshape(1, HIDDEN)
    b3r = b3.reshape(1, NUM_LAYERS)

    logits, idx = pl.pallas_call(
        _router_kernel,
        grid=(GRID,),
        in_specs=[
            pl.BlockSpec((B, S_BLK, D_MODEL),
                         lambda i: (0, jnp.minimum(i, NS - 1), 0)),
            pl.BlockSpec((H_BLK, D_MODEL),
                         lambda i: (jnp.clip(i - NS, 0, NH - 1), 0)),
            pl.BlockSpec((1, 1, H_BLK),
                         lambda i: (jnp.clip(i - NS, 0, NH - 1), 0, 0)),
            pl.BlockSpec((HIDDEN, H_BLK),
                         lambda i: (0, jnp.clip(i - NS, 0, NH - 1))),
            pl.BlockSpec((1, HIDDEN), lambda i: (0, 0)),
            pl.BlockSpec((NUM_LAYERS, HIDDEN), lambda i: (0, 0)),
            pl.BlockSpec((1, NUM_LAYERS), lambda i: (0, 0)),
        ],
        out_specs=[
            pl.BlockSpec((B, NUM_LAYERS), lambda i: (0, 0)),
            pl.BlockSpec((B, 1), lambda i: (0, 0)),
        ],
        out_shape=[
            jax.ShapeDtypeStruct((B, NUM_LAYERS), jnp.float32),
            jax.ShapeDtypeStruct((B, 1), jnp.int32),
        ],
        scratch_shapes=[
            pltpu.VMEM((B, D_MODEL), jnp.float32),
            pltpu.VMEM((B, HIDDEN), jnp.float32),
        ],
        compiler_params=pltpu.CompilerParams(
            dimension_semantics=("arbitrary",)),
    )(x, W1, b1r, W2, b2r, W3, b3r)

    return (idx.reshape(B), logits)
